# baseline (speedup 1.0000x reference)
import functools

import jax
import jax.numpy as jnp
from jax import lax
from jax.experimental import pallas as pl
from jax.experimental.pallas import tpu as pltpu
from jax.experimental.pallas import tpu_sc as plsc

EMB_DIM = 32
BATCH = 16384
NUM_ROWS = 100002

NUM_CORES = 2
NUM_SUBCORES = 16
NUM_WORKERS = NUM_CORES * NUM_SUBCORES
LANES = 16

OUT_CHUNK = 4096
N_CHUNKS = BATCH // OUT_CHUNK


@functools.partial(
    pl.kernel,
    out_type=jax.ShapeDtypeStruct((EMB_DIM, BATCH), jnp.float32),
    mesh=plsc.VectorSubcoreMesh(core_axis_name="c", subcore_axis_name="s"),
    scratch_types=[
        pltpu.VMEM((NUM_ROWS,), jnp.float32),
        pltpu.VMEM((BATCH,), jnp.int32),
        pltpu.VMEM((OUT_CHUNK,), jnp.float32),
        pltpu.VMEM((OUT_CHUNK,), jnp.float32),
        pltpu.SemaphoreType.DMA,
        pltpu.SemaphoreType.DMA,
    ],
    compiler_params=pltpu.CompilerParams(needs_layout_passes=False),
)
def _gather_kernel(
    table_hbm, idx_hbm, out_hbm, row_v, idx_v, out_a, out_b, sem, osem
):
    f = lax.axis_index("s") * NUM_CORES + lax.axis_index("c")
    row_cp = pltpu.async_copy(table_hbm.at[f], row_v, sem)
    idx_cp = pltpu.async_copy(idx_hbm, idx_v, sem)
    idx_cp.wait()
    row_cp.wait()

    out_copies = []
    for oc in range(N_CHUNKS):
        buf = (out_a, out_b)[oc % 2]
        if oc >= 2:
            out_copies[oc - 2].wait()

        depth = 4
        for g0 in range(0, OUT_CHUNK // LANES, depth):
            ivs = [
                idx_v[pl.ds(oc * OUT_CHUNK + (g0 + d) * LANES, LANES)]
                for d in range(depth)
            ]
            vals = [plsc.load_gather(row_v, [iv]) for iv in ivs]
            for d in range(depth):
                buf[pl.ds((g0 + d) * LANES, LANES)] = vals[d]

        out_copies.append(
            pltpu.async_copy(
                buf, out_hbm.at[f, pl.ds(oc * OUT_CHUNK, OUT_CHUNK)], osem
            )
        )
    for cp in out_copies[-2:]:
        cp.wait()


def kernel(x, W0):
    idx = x.reshape(BATCH).astype(jnp.int32)
    out_t = _gather_kernel(W0.T, idx)
    return out_t.T


# speedup vs baseline: 1.1778x; 1.1778x over previous
import functools

import jax
import jax.numpy as jnp
from jax import lax
from jax.experimental import pallas as pl
from jax.experimental.pallas import tpu as pltpu
from jax.experimental.pallas import tpu_sc as plsc

EMB_DIM = 32
BATCH = 16384
NUM_ROWS = 100002

NUM_CORES = 2
NUM_SUBCORES = 16
NUM_WORKERS = NUM_CORES * NUM_SUBCORES
LANES = 16

OUT_CHUNK = 4096
N_CHUNKS = BATCH // OUT_CHUNK


@functools.partial(
    pl.kernel,
    out_type=jax.ShapeDtypeStruct((EMB_DIM, BATCH), jnp.float32),
    mesh=plsc.VectorSubcoreMesh(core_axis_name="c", subcore_axis_name="s"),
    scratch_types=[
        pltpu.VMEM((NUM_ROWS,), jnp.float32),
        pltpu.VMEM((BATCH,), jnp.int32),
        pltpu.VMEM((OUT_CHUNK,), jnp.float32),
        pltpu.VMEM((OUT_CHUNK,), jnp.float32),
        pltpu.SemaphoreType.DMA,
        pltpu.SemaphoreType.DMA,
    ],
    compiler_params=pltpu.CompilerParams(needs_layout_passes=False),
)
def _gather_kernel(
    table_hbm, idx_hbm, out_hbm, row_v, idx_v, out_a, out_b, sem, osem
):
    f = lax.axis_index("s") * NUM_CORES + lax.axis_index("c")
    row_cp = pltpu.async_copy(table_hbm.at[f], row_v, sem)
    idx_cp = pltpu.async_copy(idx_hbm, idx_v, sem)
    idx_cp.wait()
    row_cp.wait()

    out_copies = []
    for oc in range(N_CHUNKS):
        buf = (out_a, out_b)[oc % 2]
        if oc >= 2:
            out_copies[oc - 2].wait()

        depth = 8
        span = LANES * depth

        @plsc.parallel_loop(0, OUT_CHUNK // span, 1, unroll=2)
        def _(b):
            ivs = [
                idx_v[pl.ds(oc * OUT_CHUNK + b * span + d * LANES, LANES)]
                for d in range(depth)
            ]
            vals = [plsc.load_gather(row_v, [iv]) for iv in ivs]
            for d in range(depth):
                buf[pl.ds(b * span + d * LANES, LANES)] = vals[d]

        out_copies.append(
            pltpu.async_copy(
                buf, out_hbm.at[f, pl.ds(oc * OUT_CHUNK, OUT_CHUNK)], osem
            )
        )
    for cp in out_copies[-2:]:
        cp.wait()


def kernel(x, W0):
    idx = x.reshape(BATCH).astype(jnp.int32)
    out_t = _gather_kernel(W0.T, idx)
    return out_t.T


# speedup vs baseline: 1.1959x; 1.0153x over previous
import functools

import jax
import jax.numpy as jnp
from jax import lax
from jax.experimental import pallas as pl
from jax.experimental.pallas import tpu as pltpu
from jax.experimental.pallas import tpu_sc as plsc

EMB_DIM = 32
BATCH = 16384
NUM_ROWS = 100002

NUM_CORES = 2
NUM_SUBCORES = 16
NUM_WORKERS = NUM_CORES * NUM_SUBCORES
LANES = 16

OUT_CHUNK = 4096
N_CHUNKS = BATCH // OUT_CHUNK


@functools.partial(
    pl.kernel,
    out_type=jax.ShapeDtypeStruct((EMB_DIM, BATCH), jnp.float32),
    mesh=plsc.VectorSubcoreMesh(core_axis_name="c", subcore_axis_name="s"),
    scratch_types=[
        pltpu.VMEM((NUM_ROWS,), jnp.float32),
        pltpu.VMEM((BATCH,), jnp.int32),
        pltpu.VMEM((OUT_CHUNK,), jnp.float32),
        pltpu.VMEM((OUT_CHUNK,), jnp.float32),
        pltpu.SemaphoreType.DMA,
        pltpu.SemaphoreType.DMA,
    ],
    compiler_params=pltpu.CompilerParams(
        needs_layout_passes=False, skip_device_barrier=True
    ),
)
def _gather_kernel(
    table_hbm, idx_hbm, out_hbm, row_v, idx_v, out_a, out_b, sem, osem
):
    f = lax.axis_index("s") * NUM_CORES + lax.axis_index("c")
    row_cp = pltpu.async_copy(table_hbm.at[f], row_v, sem)
    idx_cp = pltpu.async_copy(idx_hbm, idx_v, sem)
    idx_cp.wait()
    row_cp.wait()

    out_copies = []
    for oc in range(N_CHUNKS):
        buf = (out_a, out_b)[oc % 2]
        if oc >= 2:
            out_copies[oc - 2].wait()

        depth = 8
        span = LANES * depth

        @plsc.parallel_loop(0, OUT_CHUNK // span, 1, unroll=2)
        def _(b):
            ivs = [
                idx_v[pl.ds(oc * OUT_CHUNK + b * span + d * LANES, LANES)]
                for d in range(depth)
            ]
            vals = [plsc.load_gather(row_v, [iv]) for iv in ivs]
            for d in range(depth):
                buf[pl.ds(b * span + d * LANES, LANES)] = vals[d]

        out_copies.append(
            pltpu.async_copy(
                buf, out_hbm.at[f, pl.ds(oc * OUT_CHUNK, OUT_CHUNK)], osem
            )
        )
    for cp in out_copies[-2:]:
        cp.wait()


def kernel(x, W0):
    idx = x.reshape(BATCH).astype(jnp.int32)
    out_t = _gather_kernel(W0.T, idx)
    return out_t.T
